# baseline (speedup 1.0000x reference)
import functools

import jax
import jax.numpy as jnp
from jax import lax
from jax.experimental import pallas as pl
from jax.experimental.pallas import tpu as pltpu
from jax.experimental.pallas import tpu_sc as plsc

NF = 26
VOCAB = 1000
ED = 64
NCONT = 13
BATCH = 4096
EPS = 1e-5

NC, NS, L = 2, 16, 16
NW = NC * NS
ROWS_W = BATCH // NW
STEP = ROWS_W

BT = 256
D_FEAT = NF * ED


def _gather_body(tab_hbm, idx_hbm, out_hbm, idxv, rows, gsem, osem):
    wid = lax.axis_index("s") * NC + lax.axis_index("c")
    pltpu.sync_copy(idx_hbm.at[wid], idxv)
    for f in range(NF):
        for g in range(STEP // L):
            c = g * L
            idxv[f, pl.ds(c, L)] = idxv[f, pl.ds(c, L)] + (f * VOCAB)
    gathers = [
        pltpu.async_copy(
            tab_hbm.at[idxv.at[f]],
            rows.at[pl.ds(f * STEP, STEP)],
            gsem,
        )
        for f in range(NF)
    ]
    outs = []
    for f in range(NF):
        gathers[f].wait()
        outs.append(
            pltpu.async_copy(
                rows.at[pl.ds(f * STEP, STEP)],
                out_hbm.at[pl.ds(wid * ROWS_W, ROWS_W), pl.ds(f * ED, ED)],
                osem,
            )
        )
    for cp in outs:
        cp.wait()


def _sc_gather(tab, idx3):
    mesh = plsc.VectorSubcoreMesh(
        core_axis_name="c", subcore_axis_name="s", num_cores=NC, num_subcores=NS
    )
    f = pl.kernel(
        _gather_body,
        out_type=jax.ShapeDtypeStruct((BATCH, D_FEAT), jnp.bfloat16),
        mesh=mesh,
        scratch_types=[
            pltpu.VMEM((NF, STEP), jnp.int32),
            pltpu.VMEM((NF * STEP, ED), jnp.bfloat16),
            pltpu.SemaphoreType.DMA,
            pltpu.SemaphoreType.DMA,
        ],
        compiler_params=pltpu.CompilerParams(use_tc_tiling_on_sc=False),
    )
    return f(tab, idx3)


def _mlp_body(
    xf, xc, g0, be0, w1f, w1c, b1, g1, be1, w2, b2, g2, be2, w3, b3, g3, be3,
    wout, bout, out
):
    inv = 1.0 / jnp.sqrt(jnp.float32(1.0) + EPS)
    dot = functools.partial(lax.dot_general, preferred_element_type=jnp.float32)
    ct = (((1,), (1,)), ((), ()))
    xcb = ((xc[...] * inv) * g0[...] + be0[...]).astype(jnp.bfloat16)
    h = dot(xf[...], w1f[...], ct) + dot(xcb, w1c[...], ct)
    h = jnp.maximum(h + b1[...], 0.0)
    h = ((h * inv) * g1[...] + be1[...]).astype(jnp.bfloat16)
    h = jnp.maximum(dot(h, w2[...], ct) + b2[...], 0.0)
    h = ((h * inv) * g2[...] + be2[...]).astype(jnp.bfloat16)
    h = jnp.maximum(dot(h, w3[...], ct) + b3[...], 0.0)
    h = ((h * inv) * g3[...] + be3[...]).astype(jnp.bfloat16)
    out[...] = dot(wout[...], h, ct) + bout[0]


def _row(v):
    return v.reshape(1, -1)


def _full_spec(a):
    return pl.BlockSpec(a.shape, lambda i: (0, 0))


def kernel(x_categories_tensor101, x_continuous_tensor101, emb_tables, bn0_gamma,
           bn0_beta, W1, b1, g1, be1, W2, b2, g2, be2, W3, b3, g3, be3, Wout, bout):
    bf = jnp.bfloat16
    tab = emb_tables.reshape(NF * VOCAB, ED).astype(bf)
    idx3 = (
        x_categories_tensor101.astype(jnp.int32)
        .reshape(NW, ROWS_W, NF)
        .transpose(0, 2, 1)
    )
    xf = _sc_gather(tab, idx3)
    xc = x_continuous_tensor101
    params = [
        _row(bn0_gamma), _row(bn0_beta),
        W1[:, :D_FEAT].astype(bf), W1[:, D_FEAT:].astype(bf),
        _row(b1), _row(g1), _row(be1),
        W2.astype(bf), _row(b2), _row(g2), _row(be2),
        W3.astype(bf), _row(b3), _row(g3), _row(be3),
        Wout.astype(bf),
    ]
    out = pl.pallas_call(
        _mlp_body,
        grid=(BATCH // BT,),
        in_specs=[
            pl.BlockSpec((BT, D_FEAT), lambda i: (i, 0)),
            pl.BlockSpec((BT, NCONT), lambda i: (i, 0)),
        ] + [_full_spec(p) for p in params]
          + [pl.BlockSpec(memory_space=pltpu.SMEM)],
        out_specs=pl.BlockSpec((1, BT), lambda i: (0, i)),
        out_shape=jax.ShapeDtypeStruct((1, BATCH), jnp.float32),
    )(xf, xc, *params, bout)
    return out.reshape(BATCH, 1)


# speedup vs baseline: 1.3905x; 1.3905x over previous
import functools

import jax
import jax.numpy as jnp
from jax import lax
from jax.experimental import pallas as pl
from jax.experimental.pallas import tpu as pltpu
from jax.experimental.pallas import tpu_sc as plsc

NF = 26
VOCAB = 1000
ED = 64
NCONT = 13
BATCH = 4096
EPS = 1e-5

NC, NS, L = 2, 16, 16
NW = NC * NS
ROWS_W = BATCH // NW
STEP = ROWS_W
HALF = NF // 2

BT = 256
D_FEAT = NF * ED
KT = D_FEAT // 128


def _gather_body(tab_hbm, idx_hbm, out_hbm, idxv, rows, gsem, osem):
    wid = lax.axis_index("s") * NC + lax.axis_index("c")
    pltpu.sync_copy(idx_hbm.at[wid], idxv)
    for f in range(NF):
        for g in range(STEP // L):
            c = g * L
            idxv[f, pl.ds(c, L)] = idxv[f, pl.ds(c, L)] + (f * VOCAB)
    row0 = wid * ROWS_W
    for h in range(2):
        gathers = [
            pltpu.async_copy(
                tab_hbm.at[idxv.at[h * HALF + t]],
                rows.at[pl.ds(t * STEP, STEP)],
                gsem,
            )
            for t in range(HALF)
        ]
        outs = []
        for t in range(HALF):
            f = h * HALF + t
            gathers[t].wait()
            outs.append(
                pltpu.async_copy(
                    rows.at[pl.ds(t * STEP, STEP)],
                    out_hbm.at[f // 2, pl.ds(row0, ROWS_W),
                               pl.ds((f % 2) * ED, ED)],
                    osem,
                )
            )
        for cp in outs:
            cp.wait()


def _sc_gather(tab, idx3):
    mesh = plsc.VectorSubcoreMesh(
        core_axis_name="c", subcore_axis_name="s", num_cores=NC, num_subcores=NS
    )
    f = pl.kernel(
        _gather_body,
        out_type=jax.ShapeDtypeStruct((KT, BATCH, 128), jnp.float32),
        mesh=mesh,
        scratch_types=[
            pltpu.VMEM((NF, STEP), jnp.int32),
            pltpu.VMEM((HALF * STEP, ED), jnp.float32),
            pltpu.SemaphoreType.DMA,
            pltpu.SemaphoreType.DMA,
        ],
        compiler_params=pltpu.CompilerParams(use_tc_tiling_on_sc=False),
    )
    return f(tab, idx3)


def _mlp_body(
    xf, xc, g0, be0, w1f, w1c, b1, g1, be1, w2, b2, g2, be2, w3, b3, g3, be3,
    wout, bout, out
):
    inv = 1.0 / jnp.sqrt(jnp.float32(1.0) + EPS)
    dot = functools.partial(lax.dot_general, preferred_element_type=jnp.float32)
    ct = (((1,), (1,)), ((), ()))
    bf = jnp.bfloat16
    xcb = ((xc[...] * inv) * g0[...] + be0[...]).astype(bf)
    h = dot(xcb, w1c[...], ct)
    for c in range(KT):
        h += dot(xf[c].astype(bf), w1f[c], ct)
    h = jnp.maximum(h + b1[...], 0.0)
    h = ((h * inv) * g1[...] + be1[...]).astype(bf)
    h = jnp.maximum(dot(h, w2[...], ct) + b2[...], 0.0)
    h = ((h * inv) * g2[...] + be2[...]).astype(bf)
    h = jnp.maximum(dot(h, w3[...], ct) + b3[...], 0.0)
    h = ((h * inv) * g3[...] + be3[...]).astype(bf)
    out[...] = dot(wout[...], h, ct) + bout[0]


def _row(v):
    return v.reshape(1, -1)


def _full_spec(a):
    return pl.BlockSpec(a.shape, lambda i: tuple(0 for _ in a.shape))


def kernel(x_categories_tensor101, x_continuous_tensor101, emb_tables, bn0_gamma,
           bn0_beta, W1, b1, g1, be1, W2, b2, g2, be2, W3, b3, g3, be3, Wout, bout):
    bf = jnp.bfloat16
    tab = emb_tables.reshape(NF * VOCAB, ED)
    idx3 = (
        x_categories_tensor101.astype(jnp.int32)
        .reshape(NW, ROWS_W, NF)
        .transpose(0, 2, 1)
    )
    xf = _sc_gather(tab, idx3)
    xc = x_continuous_tensor101
    w1f = W1[:, :D_FEAT].reshape(-1, KT, 128).transpose(1, 0, 2).astype(bf)
    params = [
        _row(bn0_gamma), _row(bn0_beta),
        w1f, W1[:, D_FEAT:].astype(bf),
        _row(b1), _row(g1), _row(be1),
        W2.astype(bf), _row(b2), _row(g2), _row(be2),
        W3.astype(bf), _row(b3), _row(g3), _row(be3),
        Wout.astype(bf),
    ]
    out = pl.pallas_call(
        _mlp_body,
        grid=(BATCH // BT,),
        in_specs=[
            pl.BlockSpec((KT, BT, 128), lambda i: (0, i, 0)),
            pl.BlockSpec((BT, NCONT), lambda i: (i, 0)),
        ] + [_full_spec(p) for p in params]
          + [pl.BlockSpec(memory_space=pltpu.SMEM)],
        out_specs=pl.BlockSpec((1, BT), lambda i: (0, i)),
        out_shape=jax.ShapeDtypeStruct((1, BATCH), jnp.float32),
    )(xf, xc, *params, bout)
    return out.reshape(BATCH, 1)


# speedup vs baseline: 1.6354x; 1.1761x over previous
import functools

import jax
import jax.numpy as jnp
from jax import lax
from jax.experimental import pallas as pl
from jax.experimental.pallas import tpu as pltpu
from jax.experimental.pallas import tpu_sc as plsc

NF = 26
VOCAB = 1000
ED = 64
NCONT = 13
BATCH = 4096
EPS = 1e-5

NC, NS, L = 2, 16, 16
NW = NC * NS
ROWS_W = BATCH // NW
STEP = ROWS_W
HALF = NF // 2

BT = 256
D_FEAT = NF * ED
KT = D_FEAT // 128


def _gather_body(tab_hbm, idx_hbm, out_hbm, idxv, rows, gsem, osem):
    wid = lax.axis_index("s") * NC + lax.axis_index("c")
    pltpu.sync_copy(idx_hbm.at[wid], idxv)
    for f in range(NF):
        for g in range(STEP // L):
            c = g * L
            idxv[f, pl.ds(c, L)] = idxv[f, pl.ds(c, L)] + (f * VOCAB)
    row0 = wid * ROWS_W
    for h in range(2):
        gathers = [
            pltpu.async_copy(
                tab_hbm.at[idxv.at[h * HALF + t]],
                rows.at[pl.ds(t * STEP, STEP)],
                gsem,
            )
            for t in range(HALF)
        ]
        outs = []
        for t in range(HALF):
            f = h * HALF + t
            gathers[t].wait()
            outs.append(
                pltpu.async_copy(
                    rows.at[pl.ds(t * STEP, STEP)],
                    out_hbm.at[f // 2, pl.ds(row0, ROWS_W),
                               pl.ds((f % 2) * ED, ED)],
                    osem,
                )
            )
        for cp in outs:
            cp.wait()


def _sc_gather(tab, idx3):
    mesh = plsc.VectorSubcoreMesh(
        core_axis_name="c", subcore_axis_name="s", num_cores=NC, num_subcores=NS
    )
    f = pl.kernel(
        _gather_body,
        out_type=jax.ShapeDtypeStruct((KT, BATCH, 128), jnp.float32),
        mesh=mesh,
        scratch_types=[
            pltpu.VMEM((NF, STEP), jnp.int32),
            pltpu.VMEM((HALF * STEP, ED), jnp.float32),
            pltpu.SemaphoreType.DMA,
            pltpu.SemaphoreType.DMA,
        ],
        compiler_params=pltpu.CompilerParams(use_tc_tiling_on_sc=False),
    )
    return f(tab, idx3)


def _mlp_body(
    xf, xc, g0, be0, w1f, w1c, b1, g1, be1, w2, b2, g2, be2, w3, b3, g3, be3,
    wout, bout, out
):
    inv = 1.0 / jnp.sqrt(jnp.float32(1.0) + EPS)
    dot = functools.partial(lax.dot_general, preferred_element_type=jnp.float32)
    ct = (((1,), (1,)), ((), ()))
    bf = jnp.bfloat16
    xcb = ((xc[...] * inv) * g0[...] + be0[...]).astype(bf)
    x2 = jnp.concatenate([xf[c] for c in range(KT)], axis=1).astype(bf)
    h = dot(x2, w1f[...], ct) + dot(xcb, w1c[...], ct)
    h = jnp.maximum(h + b1[...], 0.0)
    h = ((h * inv) * g1[...] + be1[...]).astype(bf)
    h = jnp.maximum(dot(h, w2[...], ct) + b2[...], 0.0)
    h = ((h * inv) * g2[...] + be2[...]).astype(bf)
    h = jnp.maximum(dot(h, w3[...], ct) + b3[...], 0.0)
    h = ((h * inv) * g3[...] + be3[...]).astype(bf)
    out[...] = dot(wout[...], h, ct) + bout[0]


def _row(v):
    return v.reshape(1, -1)


def _full_spec(a):
    return pl.BlockSpec(a.shape, lambda i: tuple(0 for _ in a.shape))


def kernel(x_categories_tensor101, x_continuous_tensor101, emb_tables, bn0_gamma,
           bn0_beta, W1, b1, g1, be1, W2, b2, g2, be2, W3, b3, g3, be3, Wout, bout):
    bf = jnp.bfloat16
    tab = emb_tables.reshape(NF * VOCAB, ED)
    idx3 = (
        x_categories_tensor101.astype(jnp.int32)
        .reshape(NW, ROWS_W, NF)
        .transpose(0, 2, 1)
    )
    xf = _sc_gather(tab, idx3)
    xc = x_continuous_tensor101
    w1f = W1[:, :D_FEAT].astype(bf)
    params = [
        _row(bn0_gamma), _row(bn0_beta),
        w1f, W1[:, D_FEAT:].astype(bf),
        _row(b1), _row(g1), _row(be1),
        W2.astype(bf), _row(b2), _row(g2), _row(be2),
        W3.astype(bf), _row(b3), _row(g3), _row(be3),
        Wout.astype(bf),
    ]
    out = pl.pallas_call(
        _mlp_body,
        grid=(BATCH // BT,),
        in_specs=[
            pl.BlockSpec((KT, BT, 128), lambda i: (0, i, 0)),
            pl.BlockSpec((BT, NCONT), lambda i: (i, 0)),
        ] + [_full_spec(p) for p in params]
          + [pl.BlockSpec(memory_space=pltpu.SMEM)],
        out_specs=pl.BlockSpec((1, BT), lambda i: (0, i)),
        out_shape=jax.ShapeDtypeStruct((1, BATCH), jnp.float32),
    )(xf, xc, *params, bout)
    return out.reshape(BATCH, 1)
